# baseline (speedup 1.0000x reference)
import jax
import jax.numpy as jnp
from jax.experimental import pallas as pl
from jax.experimental.pallas import tpu as pltpu

N = 10000
D_IN = 128
D_OUT = 128
BM = 200
NBUF = 4
NSTEPS = N // BM


def _adj_copy(adj_hbm, bufs_ref, sems, i, slot):
    return pltpu.make_async_copy(
        adj_hbm.at[pl.ds(i * BM, BM), :],
        bufs_ref.at[pl.ds(slot * BM, BM), :],
        sems.at[slot],
    )


def _out_copy(ostage_ref, out_hbm, osems, i, oslot):
    return pltpu.make_async_copy(
        ostage_ref.at[pl.ds(oslot * BM, BM), :],
        out_hbm.at[pl.ds(i * BM, BM), :],
        osems.at[oslot],
    )


def _gcn_kernel(x_hbm, w_ref, b_ref, adj_hbm, out_hbm,
                x_ref, h_ref, bufs_ref, ostage_ref, xsem, sems, osems):
    for k in range(NBUF):
        _adj_copy(adj_hbm, bufs_ref, sems, k, k).start()
    pltpu.make_async_copy(x_hbm, x_ref, xsem).start()
    pltpu.make_async_copy(x_hbm, x_ref, xsem).wait()
    h_ref[...] = jax.lax.dot_general(
        x_ref[...], w_ref[...],
        (((1,), (1,)), ((), ())),
        preferred_element_type=jnp.float32,
    ) + b_ref[...]

    def body(i, _):
        slot = jax.lax.rem(i, NBUF)
        oslot = jax.lax.rem(i, 2)
        _adj_copy(adj_hbm, bufs_ref, sems, i, slot).wait()

        @pl.when(i >= 2)
        def _():
            _out_copy(ostage_ref, out_hbm, osems, i - 2, oslot).wait()

        ostage_ref[pl.ds(oslot * BM, BM), :] = jnp.dot(
            bufs_ref[pl.ds(slot * BM, BM), :], h_ref[...],
            preferred_element_type=jnp.float32,
        )
        _out_copy(ostage_ref, out_hbm, osems, i, oslot).start()

        @pl.when(i + NBUF < NSTEPS)
        def _():
            _adj_copy(adj_hbm, bufs_ref, sems, i + NBUF, slot).start()

        return 0

    jax.lax.fori_loop(0, NSTEPS, body, 0)
    _out_copy(ostage_ref, out_hbm, osems, NSTEPS - 2, 0).wait()
    _out_copy(ostage_ref, out_hbm, osems, NSTEPS - 1, 1).wait()


@jax.jit
def kernel(x, adj, W, b):
    out = pl.pallas_call(
        _gcn_kernel,
        in_specs=[
            pl.BlockSpec(memory_space=pl.ANY),
            pl.BlockSpec((D_OUT, D_IN), lambda: (0, 0)),
            pl.BlockSpec((1, D_OUT), lambda: (0, 0)),
            pl.BlockSpec(memory_space=pl.ANY),
        ],
        out_specs=pl.BlockSpec(memory_space=pl.ANY),
        out_shape=jax.ShapeDtypeStruct((N, D_OUT), jnp.float32),
        scratch_shapes=[
            pltpu.VMEM((N, D_IN), jnp.float32),
            pltpu.VMEM((N, D_OUT), jnp.float32),
            pltpu.VMEM((NBUF * BM, N), jnp.float32),
            pltpu.VMEM((2 * BM, D_OUT), jnp.float32),
            pltpu.SemaphoreType.DMA,
            pltpu.SemaphoreType.DMA((NBUF,)),
            pltpu.SemaphoreType.DMA((2,)),
        ],
    )(x, W, b.reshape(1, D_OUT), adj)
    return out


# speedup vs baseline: 1.0215x; 1.0215x over previous
import jax
import jax.numpy as jnp
from jax.experimental import pallas as pl
from jax.experimental.pallas import tpu as pltpu

N = 10000
D_IN = 128
D_OUT = 128
BM = 200
NBUF = 4
NSTEPS = N // BM


def _gcn_kernel(x_ref, w_ref, b_ref, adj_hbm, out_hbm, h_ref):
    h_ref[...] = jax.lax.dot_general(
        x_ref[...], w_ref[...],
        (((1,), (1,)), ((), ())),
        preferred_element_type=jnp.float32,
    ) + b_ref[...]

    def inner(adj_blk, out_blk):
        out_blk[...] = jnp.dot(
            adj_blk[...], h_ref[...], preferred_element_type=jnp.float32
        )

    pltpu.emit_pipeline(
        inner,
        grid=(NSTEPS,),
        in_specs=[
            pl.BlockSpec((BM, N), lambda i: (i, 0),
                         pipeline_mode=pl.Buffered(buffer_count=NBUF)),
        ],
        out_specs=[pl.BlockSpec((BM, D_OUT), lambda i: (i, 0))],
    )(adj_hbm, out_hbm)


@jax.jit
def kernel(x, adj, W, b):
    out = pl.pallas_call(
        _gcn_kernel,
        in_specs=[
            pl.BlockSpec((N, D_IN), lambda: (0, 0)),
            pl.BlockSpec((D_OUT, D_IN), lambda: (0, 0)),
            pl.BlockSpec((1, D_OUT), lambda: (0, 0)),
            pl.BlockSpec(memory_space=pl.ANY),
        ],
        out_specs=pl.BlockSpec(memory_space=pl.ANY),
        out_shape=jax.ShapeDtypeStruct((N, D_OUT), jnp.float32),
        scratch_shapes=[pltpu.VMEM((N, D_OUT), jnp.float32)],
    )(x, W, b.reshape(1, D_OUT), adj)
    return out


# speedup vs baseline: 1.0377x; 1.0159x over previous
import jax
import jax.numpy as jnp
from jax.experimental import pallas as pl
from jax.experimental.pallas import tpu as pltpu

N = 10000
D_IN = 128
D_OUT = 128
BM = 400


def _gcn_kernel(x_ref, w_ref, b_ref, adj_ref, out_ref, h_ref):
    @pl.when(pl.program_id(0) == 0)
    def _():
        h_ref[...] = jax.lax.dot_general(
            x_ref[...], w_ref[...],
            (((1,), (1,)), ((), ())),
            preferred_element_type=jnp.float32,
        ) + b_ref[...]

    out_ref[...] = jnp.dot(
        adj_ref[...], h_ref[...], preferred_element_type=jnp.float32
    )


@jax.jit
def kernel(x, adj, W, b):
    out = pl.pallas_call(
        _gcn_kernel,
        grid=(N // BM,),
        in_specs=[
            pl.BlockSpec((N, D_IN), lambda i: (0, 0)),
            pl.BlockSpec((D_OUT, D_IN), lambda i: (0, 0)),
            pl.BlockSpec((1, D_OUT), lambda i: (0, 0)),
            pl.BlockSpec((BM, N), lambda i: (i, 0)),
        ],
        out_specs=pl.BlockSpec((BM, D_OUT), lambda i: (i, 0)),
        out_shape=jax.ShapeDtypeStruct((N, D_OUT), jnp.float32),
        scratch_shapes=[pltpu.VMEM((N, D_OUT), jnp.float32)],
    )(x, W, b.reshape(1, D_OUT), adj)
    return out
